# baseline (speedup 1.0000x reference)
import functools

import jax
import jax.numpy as jnp
from jax import lax
from jax.experimental import pallas as pl
from jax.experimental.pallas import tpu as pltpu
from jax.experimental.pallas import tpu_sc as plsc

_B = 2
_N = 2048
_D = 128
_K = 16
_RT = 256
_NT = _N // _RT

_NC, _NS, _L = 2, 16, 16
_NW = _NC * _NS
_PROWS = _N // 2
_RPW = _B * _PROWS // _NW
_CH = 8
_NCH = _RPW // _CH

_LOG2_COEF = (
    -3.0283174810372375, 6.065830143177264, -5.2641104770701075,
    3.218832837050299, -1.2342631730323361, 0.26685882285942003,
    -0.024825606614202734,
)
_HALF_LN2 = 0.34657359027997264


def _cdist_body(x_ref, xt_ref, d2_ref, sq_ref):
    t = pl.program_id(0)
    xr = x_ref[0]
    xt = xt_ref[0]

    @pl.when(t % _NT == 0)
    def _():
        sq_ref[...] = jnp.sum(xt * xt, axis=0, keepdims=True)

    sq_r = jnp.sum(xr * xr, axis=1, keepdims=True)
    g = jax.lax.dot_general(
        xr, xt, (((1,), (0,)), ((), ())),
        preferred_element_type=jnp.float32,
        precision=jax.lax.Precision.DEFAULT)
    d2 = sq_r + sq_ref[...] - 2.0 * g
    rows = (t % _NT) * _RT + jax.lax.broadcasted_iota(
        jnp.int32, (_RT, _N), 0)
    cols = jax.lax.broadcasted_iota(jnp.int32, (_RT, _N), 1)
    d2w = jnp.where(rows == cols, jnp.inf, jnp.maximum(d2, 1e-12))
    half = _RT // 2
    top = jax.lax.bitcast_convert_type(d2w[:half], jnp.uint32)
    bot = jax.lax.bitcast_convert_type(d2w[half:], jnp.uint32)
    rnd = jnp.uint32(0x8000)
    hi_mask = jnp.uint32(0xFFFF0000)
    d2_ref[...] = ((top + rnd) & hi_mask) | ((bot + rnd) >> 16)


def _cdist(X, xt):
    return pl.pallas_call(
        _cdist_body,
        grid=(_B * _NT,),
        in_specs=[
            pl.BlockSpec((1, _RT, _D), lambda t: (t // _NT, t % _NT, 0)),
            pl.BlockSpec((1, _D, _N), lambda t: (t // _NT, 0, 0)),
        ],
        out_specs=pl.BlockSpec((_RT // 2, _N), lambda t: (t, 0)),
        out_shape=jax.ShapeDtypeStruct((_B * _PROWS, _N), jnp.uint32),
        scratch_shapes=[pltpu.VMEM((1, _N), jnp.float32)],
    )(X, xt)


def _log2(q):
    bits = plsc.bitcast(q, jnp.int32)
    e = ((bits >> 23) & 0xFF) - 127
    mant = plsc.bitcast((bits & 0x7FFFFF) | 0x3F800000, jnp.float32)
    p = jnp.full((_L,), _LOG2_COEF[6], jnp.float32)
    for c in (_LOG2_COEF[5], _LOG2_COEF[4], _LOG2_COEF[3],
              _LOG2_COEF[2], _LOG2_COEF[1], _LOG2_COEF[0]):
        p = p * mant + c
    return e.astype(jnp.float32) + p


def _bcast_lane(x, lane_idx):
    idx = jnp.full((_L, 1), lane_idx, jnp.int32)
    return lax.gather(
        x, idx,
        dimension_numbers=lax.GatherDimensionNumbers(
            offset_dims=(), collapsed_slice_dims=(0,), start_index_map=(0,)),
        slice_sizes=(1,),
        mode=lax.GatherScatterMode.PROMISE_IN_BOUNDS)


def _merge16(T, v):
    cs, _ = plsc.sort_key_val(v, v, descending=True)
    m = jnp.minimum(T, cs)
    out, _ = plsc.sort_key_val(m, m)
    return out


def _process_chunk(buf, acc, lane, inf_v):
    hi_mask = jnp.full((_L,), 0xFFFF0000, jnp.uint32)

    def vstep(i, Ts):
        new = []
        for r in range(_CH):
            w = buf[r, pl.ds(i * _L, _L)]
            a = plsc.bitcast(w & hi_mask, jnp.float32)
            b = plsc.bitcast(w << 16, jnp.float32)
            new.append(_merge16(Ts[2 * r], a))
            new.append(_merge16(Ts[2 * r + 1], b))
        return tuple(new)

    Ts = lax.fori_loop(0, _N // _L, vstep, (inf_v,) * (2 * _CH))
    for r in range(2 * _CH):
        lg = _log2(jnp.maximum(Ts[r], 1e-12))
        l14 = _bcast_lane(lg, 14)
        acc = acc + jnp.where(lane <= 14, l14 - lg, 0.0)
    return acc


def _sc_body(d2_hbm, out_hbm, bufa, bufb, accv, sema, semb):
    wid = lax.axis_index("s") * _NC + lax.axis_index("c")
    row0 = wid * _RPW
    lane = lax.broadcasted_iota(jnp.int32, (_L,), 0)
    inf_v = jnp.full((_L,), jnp.inf, jnp.float32)

    def start(ch, buf, sem):
        pltpu.async_copy(d2_hbm.at[pl.ds(row0 + ch * _CH, _CH)], buf, sem)

    def wait(buf, sem):
        pltpu.make_async_copy(d2_hbm.at[pl.ds(row0, _CH)], buf, sem).wait()

    start(0, bufa, sema)
    start(1, bufb, semb)

    def pair(g, acc):
        wait(bufa, sema)
        acc = _process_chunk(bufa, acc, lane, inf_v)
        start(jnp.minimum(2 * g + 2, _NCH - 2), bufa, sema)
        wait(bufb, semb)
        acc = _process_chunk(bufb, acc, lane, inf_v)
        start(jnp.minimum(2 * g + 3, _NCH - 1), bufb, semb)
        return acc

    acc = lax.fori_loop(0, _NCH // 2, pair, jnp.zeros((_L,), jnp.float32))
    wait(bufa, sema)
    wait(bufb, semb)

    accv[...] = acc * _HALF_LN2
    pltpu.sync_copy(accv, out_hbm.at[wid])


@functools.partial(
    pl.kernel,
    out_type=jax.ShapeDtypeStruct((_NW, _L), jnp.float32),
    mesh=plsc.VectorSubcoreMesh(core_axis_name="c", subcore_axis_name="s"),
    compiler_params=pltpu.CompilerParams(needs_layout_passes=False),
    cost_estimate=pl.CostEstimate(
        flops=4 * _PROWS * _N, bytes_accessed=4 * _PROWS * _N,
        transcendentals=0),
    scratch_types=[
        pltpu.VMEM((_CH, _N), jnp.uint32),
        pltpu.VMEM((_CH, _N), jnp.uint32),
        pltpu.VMEM((_L,), jnp.float32),
        pltpu.SemaphoreType.DMA,
        pltpu.SemaphoreType.DMA,
    ],
)
def _sc_select(d2_hbm, out_hbm, bufa, bufb, accv, sema, semb):
    _sc_body(d2_hbm, out_hbm, bufa, bufb, accv, sema, semb)


def kernel(X):
    xt = jnp.swapaxes(X, 1, 2)
    d2 = _cdist(X, xt)
    parts = _sc_select(d2)
    s = parts.reshape(_B, -1).sum(axis=1)
    return (_K - 2) * _N / s


# speedup vs baseline: 1.0356x; 1.0356x over previous
import functools

import jax
import jax.numpy as jnp
from jax import lax
from jax.experimental import pallas as pl
from jax.experimental.pallas import tpu as pltpu
from jax.experimental.pallas import tpu_sc as plsc

_B = 2
_N = 2048
_D = 128
_K = 16
_RT = 256
_NT = _N // _RT

_NC, _NS, _L = 2, 16, 16
_NW = _NC * _NS
_PROWS = _N // 2
_RPW = _B * _PROWS // _NW
_CH = 4
_NCH = _RPW // _CH

_LOG2_COEF = (
    -3.0283174810372375, 6.065830143177264, -5.2641104770701075,
    3.218832837050299, -1.2342631730323361, 0.26685882285942003,
    -0.024825606614202734,
)
_HALF_LN2 = 0.34657359027997264


def _cdist_body(x_ref, xt_ref, d2_ref, sq_ref):
    t = pl.program_id(0)
    xr = x_ref[0]
    xt = xt_ref[0]

    @pl.when(t % _NT == 0)
    def _():
        sq_ref[...] = jnp.sum(xt * xt, axis=0, keepdims=True)

    sq_r = jnp.sum(xr * xr, axis=1, keepdims=True)
    g = jax.lax.dot_general(
        xr, xt, (((1,), (0,)), ((), ())),
        preferred_element_type=jnp.float32,
        precision=jax.lax.Precision.DEFAULT)
    d2 = sq_r + sq_ref[...] - 2.0 * g
    rows = (t % _NT) * _RT + jax.lax.broadcasted_iota(
        jnp.int32, (_RT, _N), 0)
    cols = jax.lax.broadcasted_iota(jnp.int32, (_RT, _N), 1)
    d2w = jnp.where(rows == cols, jnp.inf, jnp.maximum(d2, 1e-12))
    half = _RT // 2
    top = jax.lax.bitcast_convert_type(d2w[:half], jnp.uint32)
    bot = jax.lax.bitcast_convert_type(d2w[half:], jnp.uint32)
    rnd = jnp.uint32(0x8000)
    hi_mask = jnp.uint32(0xFFFF0000)
    d2_ref[...] = ((top + rnd) & hi_mask) | ((bot + rnd) >> 16)


def _cdist(X, xt):
    return pl.pallas_call(
        _cdist_body,
        grid=(_B * _NT,),
        in_specs=[
            pl.BlockSpec((1, _RT, _D), lambda t: (t // _NT, t % _NT, 0)),
            pl.BlockSpec((1, _D, _N), lambda t: (t // _NT, 0, 0)),
        ],
        out_specs=pl.BlockSpec((_RT // 2, _N), lambda t: (t, 0)),
        out_shape=jax.ShapeDtypeStruct((_B * _PROWS, _N), jnp.uint32),
        scratch_shapes=[pltpu.VMEM((1, _N), jnp.float32)],
    )(X, xt)


def _log2(q):
    bits = plsc.bitcast(q, jnp.int32)
    e = ((bits >> 23) & 0xFF) - 127
    mant = plsc.bitcast((bits & 0x7FFFFF) | 0x3F800000, jnp.float32)
    p = jnp.full((_L,), _LOG2_COEF[6], jnp.float32)
    for c in (_LOG2_COEF[5], _LOG2_COEF[4], _LOG2_COEF[3],
              _LOG2_COEF[2], _LOG2_COEF[1], _LOG2_COEF[0]):
        p = p * mant + c
    return e.astype(jnp.float32) + p


def _bcast_lane(x, lane_idx):
    idx = jnp.full((_L, 1), lane_idx, jnp.int32)
    return lax.gather(
        x, idx,
        dimension_numbers=lax.GatherDimensionNumbers(
            offset_dims=(), collapsed_slice_dims=(0,), start_index_map=(0,)),
        slice_sizes=(1,),
        mode=lax.GatherScatterMode.PROMISE_IN_BOUNDS)


def _merge16(T, v):
    cs, _ = plsc.sort_key_val(v, v, descending=True)
    m = jnp.minimum(T, cs)
    out, _ = plsc.sort_key_val(m, m)
    return out


def _process_chunk(buf, acc, lane, inf_v):
    hi_mask = jnp.full((_L,), 0xFFFF0000, jnp.uint32)

    def vstep(i, Ts):
        new = []
        for r in range(_CH):
            w = buf[r, pl.ds(i * _L, _L)]
            a = plsc.bitcast(w & hi_mask, jnp.float32)
            b = plsc.bitcast(w << 16, jnp.float32)
            new.append(_merge16(Ts[2 * r], a))
            new.append(_merge16(Ts[2 * r + 1], b))
        return tuple(new)

    Ts = lax.fori_loop(0, _N // _L, vstep, (inf_v,) * (2 * _CH))
    for r in range(2 * _CH):
        lg = _log2(jnp.maximum(Ts[r], 1e-12))
        l14 = _bcast_lane(lg, 14)
        acc = acc + jnp.where(lane <= 14, l14 - lg, 0.0)
    return acc


def _sc_body(d2_hbm, out_hbm, bufa, bufb, accv, sema, semb):
    wid = lax.axis_index("s") * _NC + lax.axis_index("c")
    row0 = wid * _RPW
    lane = lax.broadcasted_iota(jnp.int32, (_L,), 0)
    inf_v = jnp.full((_L,), jnp.inf, jnp.float32)

    def start(ch, buf, sem):
        pltpu.async_copy(d2_hbm.at[pl.ds(row0 + ch * _CH, _CH)], buf, sem)

    def wait(buf, sem):
        pltpu.make_async_copy(d2_hbm.at[pl.ds(row0, _CH)], buf, sem).wait()

    start(0, bufa, sema)
    start(1, bufb, semb)

    def pair(g, acc):
        wait(bufa, sema)
        acc = _process_chunk(bufa, acc, lane, inf_v)
        start(jnp.minimum(2 * g + 2, _NCH - 2), bufa, sema)
        wait(bufb, semb)
        acc = _process_chunk(bufb, acc, lane, inf_v)
        start(jnp.minimum(2 * g + 3, _NCH - 1), bufb, semb)
        return acc

    acc = lax.fori_loop(0, _NCH // 2, pair, jnp.zeros((_L,), jnp.float32))
    wait(bufa, sema)
    wait(bufb, semb)

    accv[...] = acc * _HALF_LN2
    pltpu.sync_copy(accv, out_hbm.at[wid])


@functools.partial(
    pl.kernel,
    out_type=jax.ShapeDtypeStruct((_NW, _L), jnp.float32),
    mesh=plsc.VectorSubcoreMesh(core_axis_name="c", subcore_axis_name="s"),
    compiler_params=pltpu.CompilerParams(needs_layout_passes=False),
    cost_estimate=pl.CostEstimate(
        flops=4 * _PROWS * _N, bytes_accessed=4 * _PROWS * _N,
        transcendentals=0),
    scratch_types=[
        pltpu.VMEM((_CH, _N), jnp.uint32),
        pltpu.VMEM((_CH, _N), jnp.uint32),
        pltpu.VMEM((_L,), jnp.float32),
        pltpu.SemaphoreType.DMA,
        pltpu.SemaphoreType.DMA,
    ],
)
def _sc_select(d2_hbm, out_hbm, bufa, bufb, accv, sema, semb):
    _sc_body(d2_hbm, out_hbm, bufa, bufb, accv, sema, semb)


def kernel(X):
    xt = jnp.swapaxes(X, 1, 2)
    d2 = _cdist(X, xt)
    parts = _sc_select(d2)
    s = parts.reshape(_B, -1).sum(axis=1)
    return (_K - 2) * _N / s
